# baseline (speedup 1.0000x reference)
import functools

import jax
import jax.numpy as jnp
from jax import lax
from jax.experimental import pallas as pl
from jax.experimental.pallas import tpu as pltpu
from jax.experimental.pallas import tpu_sc as plsc

B = 16384
IN_BITS = 18
NEURONS = 6
NB = 12
TABLE = 2 ** NB

_info = plsc.get_sparse_core_info()
NC = 1
NS = _info.num_subcores
L = _info.num_lanes
NW = NC * NS
BPW = B // NW
CHUNKS = BPW // L
CONN_PAD = 96

_mesh = plsc.VectorSubcoreMesh(core_axis_name="c", subcore_axis_name="s", num_cores=1)


@functools.partial(
    pl.kernel,
    mesh=_mesh,
    compiler_params=pltpu.CompilerParams(needs_layout_passes=False),
    out_type=jax.ShapeDtypeStruct((B * NEURONS,), jnp.float32),
    scratch_types=[
        pltpu.VMEM((IN_BITS * BPW,), jnp.int32),
        pltpu.VMEM((NEURONS * TABLE,), jnp.float32),
        pltpu.VMEM((CONN_PAD,), jnp.int32),
        pltpu.VMEM((BPW * NEURONS,), jnp.float32),
        pltpu.SemaphoreType.DMA,
    ],
)
def _ngram_kernel(
    xb_hbm, conn_hbm, mem_hbm, out_hbm, xb_v, tab_v, conn_v, out_v, sem
):
    wid = lax.axis_index("s") * NC + lax.axis_index("c")
    base = wid * BPW

    c_xb = pltpu.async_copy(
        xb_hbm.at[pl.ds(wid * IN_BITS * BPW, IN_BITS * BPW)], xb_v, sem
    )
    c_tab = pltpu.async_copy(mem_hbm, tab_v, sem)
    c_conn = pltpu.async_copy(conn_hbm, conn_v.at[pl.ds(0, NEURONS * NB)], sem)
    c_xb.wait()
    c_tab.wait()
    c_conn.wait()

    conn_rows = [conn_v[pl.ds(j * NB, L)] for j in range(NEURONS)]
    conns = [[conn_rows[j][k] for k in range(NB)] for j in range(NEURONS)]
    lane_iota = lax.iota(jnp.int32, L)

    def body(ci, carry):
        b0 = ci * L
        for j in range(NEURONS):
            acc = jnp.full((L,), j * TABLE, jnp.int32)
            for k in range(NB):
                bits = xb_v[pl.ds(conns[j][k] * BPW + b0, L)]
                acc = acc + (bits << (NB - 1 - k))
            vals = plsc.load_gather(tab_v, [acc])
            plsc.store_scatter(out_v, [(b0 + lane_iota) * NEURONS + j], vals)
        return carry

    lax.fori_loop(0, CHUNKS, body, 0)

    pltpu.sync_copy(out_v, out_hbm.at[pl.ds(base * NEURONS, BPW * NEURONS)])


def kernel(x_bits, connections, memory):
    xb_flat = (
        x_bits.astype(jnp.int32)
        .T.reshape(IN_BITS, NW, BPW)
        .transpose(1, 0, 2)
        .reshape(-1)
    )
    out = _ngram_kernel(
        xb_flat, connections.astype(jnp.int32).reshape(-1), memory.reshape(-1)
    )
    return out.reshape(B, NEURONS)


# speedup vs baseline: 1.0127x; 1.0127x over previous
import functools

import jax
import jax.numpy as jnp
from jax import lax
from jax.experimental import pallas as pl
from jax.experimental.pallas import tpu as pltpu
from jax.experimental.pallas import tpu_sc as plsc

B = 16384
IN_BITS = 18
NEURONS = 6
NB = 12
TABLE = 2 ** NB

_info = plsc.get_sparse_core_info()
NC = _info.num_cores
NS = _info.num_subcores
L = _info.num_lanes
NW = NC * NS
BPW = B // NW
CHUNKS = BPW // L
CONN_PAD = 96

_mesh = plsc.VectorSubcoreMesh(core_axis_name="c", subcore_axis_name="s")


@functools.partial(
    pl.kernel,
    mesh=_mesh,
    compiler_params=pltpu.CompilerParams(needs_layout_passes=False),
    out_type=jax.ShapeDtypeStruct((B * NEURONS,), jnp.float32),
    scratch_types=[
        pltpu.VMEM((IN_BITS * BPW,), jnp.int32),
        pltpu.VMEM((NEURONS * TABLE,), jnp.float32),
        pltpu.VMEM((CONN_PAD,), jnp.int32),
        pltpu.VMEM((BPW * NEURONS,), jnp.float32),
        pltpu.SemaphoreType.DMA,
    ],
)
def _ngram_kernel(
    xb_hbm, conn_hbm, mem_hbm, out_hbm, xb_v, tab_v, conn_v, out_v, sem
):
    wid = lax.axis_index("s") * NC + lax.axis_index("c")
    base = wid * BPW

    c_xb = pltpu.async_copy(
        xb_hbm.at[pl.ds(wid * IN_BITS * BPW, IN_BITS * BPW)], xb_v, sem
    )
    c_tab = pltpu.async_copy(mem_hbm, tab_v, sem)
    c_conn = pltpu.async_copy(conn_hbm, conn_v.at[pl.ds(0, NEURONS * NB)], sem)
    c_xb.wait()
    c_tab.wait()
    c_conn.wait()

    conn_rows = [conn_v[pl.ds(j * NB, L)] for j in range(NEURONS)]
    conns = [[conn_rows[j][k] for k in range(NB)] for j in range(NEURONS)]
    lane_iota = lax.iota(jnp.int32, L)

    def body(ci, carry):
        b0 = ci * L
        for j in range(NEURONS):
            acc = jnp.full((L,), j * TABLE, jnp.int32)
            for k in range(NB):
                bits = xb_v[pl.ds(conns[j][k] * BPW + b0, L)]
                acc = acc + (bits << (NB - 1 - k))
            vals = plsc.load_gather(tab_v, [acc])
            plsc.store_scatter(out_v, [(b0 + lane_iota) * NEURONS + j], vals)
        return carry

    lax.fori_loop(0, CHUNKS, body, 0)

    pltpu.sync_copy(out_v, out_hbm.at[pl.ds(base * NEURONS, BPW * NEURONS)])


def kernel(x_bits, connections, memory):
    xb_flat = (
        x_bits.astype(jnp.int32)
        .T.reshape(IN_BITS, NW, BPW)
        .transpose(1, 0, 2)
        .reshape(-1)
    )
    out = _ngram_kernel(
        xb_flat, connections.astype(jnp.int32).reshape(-1), memory.reshape(-1)
    )
    return out.reshape(B, NEURONS)
